# Initial kernel scaffold; baseline (speedup 1.0000x reference)
import jax
import jax.numpy as jnp
from jax.experimental import pallas as pl


def kernel(x):
    raise NotImplementedError("write your pallas kernel here")


# speedup vs baseline: 1.0540x; 1.0540x over previous
import random as _pyrandom

import jax
import jax.numpy as jnp
import numpy as np
from jax.experimental import pallas as pl
from jax.experimental.pallas import tpu as pltpu

_NUM_CENTERS = 1024
_NUM_ITERS = 10
_EPS = 1e-16
_N_POINTS = 4096
_D_CHUNK = 1024
_U_CHUNK = 512

_pyrandom.seed(42)
_INDS = np.array(_pyrandom.sample(range(_N_POINTS), _NUM_CENTERS), dtype=np.int32)


def _kmeans_body(v0_ref, x_ref, u_ref, v_ref, uscratch, dsem):
    i = pl.program_id(0)
    x = x_ref[0]
    x2 = jnp.sum(x * x, axis=-1, keepdims=True)
    v = v0_ref[0]

    uiota = jax.lax.broadcasted_iota(jnp.int32, (_U_CHUNK, _NUM_CENTERS), 1)

    def center_sq(v):
        p = v * v
        pt = p.T
        a = []
        for s in range(8):
            acc = jax.lax.slice(pt, (s, 0), (s + 1, _NUM_CENTERS))
            for r in range(1, 8):
                acc = acc + jax.lax.slice(
                    pt, (8 * r + s, 0), (8 * r + s + 1, _NUM_CENTERS)
                )
            a.append(acc)
        t1 = [a[s] + a[s + 4] for s in range(4)]
        t2 = [t1[s] + t1[s + 2] for s in range(2)]
        return (t2[0] + t2[1])[0]

    def assign(v):
        v2 = center_sq(v)
        cls = []
        for s in range(0, _N_POINTS, _D_CHUNK):
            xc = jax.lax.slice(x, (s, 0), (s + _D_CHUNK, 64))
            x2c = jax.lax.slice(x2, (s, 0), (s + _D_CHUNK, 1))
            xv = jax.lax.dot_general(
                xc, v, (((1,), (1,)), ((), ())), preferred_element_type=jnp.float32
            )
            d = jnp.maximum((x2c - 2.0 * xv) + v2[None, :], 0.0)
            cls.append(jnp.argmin(d, axis=-1, keepdims=True).astype(jnp.int32))
        return jnp.concatenate(cls, axis=0)

    def unorm(cl):
        cnt = jnp.zeros((1, _NUM_CENTERS), dtype=jnp.float32)
        chunks = []
        for s in range(0, _N_POINTS, _U_CHUNK):
            clc = jax.lax.slice(cl, (s, 0), (s + _U_CHUNK, 1))
            uc = jnp.where(uiota == clc, 1.0, 0.0).astype(jnp.float32)
            cnt = cnt + jnp.sum(uc, axis=0, keepdims=True)
            chunks.append(uc)
        den = cnt + _EPS
        return jnp.concatenate([(c + _EPS) / den for c in chunks], axis=0)

    def one_iter(_, carry):
        v, _ = carry
        cl = assign(v)
        un = unorm(cl)
        v_new = jax.lax.dot_general(
            un, x, (((0,), (0,)), ((), ())), preferred_element_type=jnp.float32
        )
        return (v_new, cl)

    cl0 = jnp.zeros((_N_POINTS, 1), dtype=jnp.int32)
    v_fin, cl_fin = jax.lax.fori_loop(0, _NUM_ITERS, one_iter, (v, cl0))

    n_chunks = _N_POINTS // _U_CHUNK
    for k in range(n_chunks):
        s = k * _U_CHUNK
        buf = k % 2
        if k >= 2:
            pltpu.make_async_copy(
                uscratch.at[buf], u_ref.at[i, pl.ds(s - 2 * _U_CHUNK, _U_CHUNK), :],
                dsem.at[buf],
            ).wait()
        clc = jax.lax.slice(cl_fin, (s, 0), (s + _U_CHUNK, 1))
        uscratch[buf] = jnp.where(uiota == clc, 1.0, 0.0).astype(jnp.float32)
        pltpu.make_async_copy(
            uscratch.at[buf], u_ref.at[i, pl.ds(s, _U_CHUNK), :], dsem.at[buf]
        ).start()
    for k in range(n_chunks - 2, n_chunks):
        s = k * _U_CHUNK
        buf = k % 2
        pltpu.make_async_copy(
            uscratch.at[buf], u_ref.at[i, pl.ds(s, _U_CHUNK), :], dsem.at[buf]
        ).wait()
    v_ref[0] = v_fin


def kernel(x):
    b = x.shape[0]
    v0 = jnp.take(x, jnp.asarray(_INDS), axis=-2)
    u, v = pl.pallas_call(
        _kmeans_body,
        grid=(b,),
        in_specs=[
            pl.BlockSpec((1, _NUM_CENTERS, 64), lambda i: (i, 0, 0)),
            pl.BlockSpec((1, _N_POINTS, 64), lambda i: (i, 0, 0)),
        ],
        out_specs=[
            pl.BlockSpec(memory_space=pltpu.MemorySpace.HBM),
            pl.BlockSpec((1, _NUM_CENTERS, 64), lambda i: (i, 0, 0)),
        ],
        out_shape=[
            jax.ShapeDtypeStruct((b, _N_POINTS, _NUM_CENTERS), jnp.float32),
            jax.ShapeDtypeStruct((b, _NUM_CENTERS, 64), jnp.float32),
        ],
        scratch_shapes=[
            pltpu.VMEM((2, _U_CHUNK, _NUM_CENTERS), jnp.float32),
            pltpu.SemaphoreType.DMA((2,)),
        ],
    )(v0, x)
    return (u, v)
